# baseline (speedup 1.0000x reference)
import functools

import jax
import jax.numpy as jnp
from jax import lax
from jax.experimental import pallas as pl
from jax.experimental.pallas import tpu as pltpu
from jax.experimental.pallas import tpu_sc as plsc

_NC = 2
_NS = 16
_C = 125
_CW = 8
_K = 8
_D4 = 4


@functools.lru_cache(maxsize=None)
def _make_agg(N, D, nch, with_counts):
    H = D // 2
    assert nch % _K == 0
    assert N % 80 == 0
    wb_rows = N // 10

    mesh = plsc.VectorSubcoreMesh(
        core_axis_name="c", subcore_axis_name="s",
        num_cores=_NC, num_subcores=_NS)

    out_type = [jax.ShapeDtypeStruct((N, D), jnp.bfloat16)]
    scratch = [
        pltpu.VMEM((nch, _C), jnp.int32),
        pltpu.VMEM((nch, _C), jnp.int32),
        pltpu.VMEM((_K, _C, H), jnp.bfloat16),
        pltpu.VMEM_SHARED((N, H), jnp.bfloat16),
    ] + [pltpu.SemaphoreType.DMA] * (2 * _K)
    if with_counts:
        out_type.append(jax.ShapeDtypeStruct((N, _CW), jnp.float32))
        scratch += [
            pltpu.VMEM((_C, _CW), jnp.float32),
            pltpu.VMEM_SHARED((N, _CW), jnp.float32),
            pltpu.SemaphoreType.DMA,
        ]

    @functools.partial(
        pl.kernel,
        out_type=tuple(out_type),
        mesh=mesh,
        scratch_types=scratch,
        compiler_params=pltpu.CompilerParams(use_tc_tiling_on_sc=False),
    )
    def agg(x2_hbm, src_hbm, dst_hbm, zd_hbm, *rest):
        if with_counts:
            (zc_hbm, ones_hbm, outs_hbm, outc_hbm,
             sidx, didx, ring, acc, *sems) = rest
            ones, cacc, csem = sems[-3:]
            sems = sems[:-3]
        else:
            (outs_hbm, sidx, didx, ring, acc, *sems) = rest
        gsem = sems[:_K]
        ssem = sems[_K:2 * _K]
        c = lax.axis_index("c")
        s = lax.axis_index("s")

        @pl.when(s == 0)
        def _():
            pltpu.sync_copy(zd_hbm, acc)
            if with_counts:
                @pl.when(c == 0)
                def _():
                    pltpu.sync_copy(zc_hbm, cacc)

        plsc.subcore_barrier()

        if with_counts:
            pltpu.sync_copy(ones_hbm, ones)

        pltpu.sync_copy(src_hbm.at[c, s], sidx)
        pltpu.sync_copy(dst_hbm.at[s], didx)

        def fire_gather(m, b):
            pltpu.async_copy(x2_hbm.at[sidx.at[m]], ring.at[b], gsem[b])

        def wait_gather(m, b):
            pltpu.make_async_copy(x2_hbm.at[sidx.at[m]], ring.at[b],
                                  gsem[b]).wait()

        def fire_scatter(m, b):
            pltpu.async_copy(ring.at[b], acc.at[didx.at[m]], ssem[b],
                             add=True)

        def wait_scatter(m, b):
            pltpu.make_async_copy(ring.at[b], acc.at[didx.at[m]],
                                  ssem[b]).wait()

        for m in range(_D4):
            fire_gather(m, m)

        @pl.loop(0, nch, step=_K)
        def _(j):
            for o in range(_K):
                jj = j + o
                wait_gather(jj, o)
                fire_scatter(jj, o)
                if with_counts:
                    @pl.when(c == 0)
                    def _():
                        pltpu.async_copy(ones, cacc.at[didx.at[jj]], csem,
                                         add=True).wait()
                ob = (o + _D4) % _K
                @pl.when(jj >= _K - _D4)
                def _():
                    wait_scatter(jj - (_K - _D4), ob)

                @pl.when(jj + _D4 < nch)
                def _():
                    fire_gather(jj + _D4, ob)

        for m in range(nch - (_K - _D4), nch):
            wait_scatter(m, m % _K)

        plsc.subcore_barrier()

        @pl.when(s < 10)
        def _():
            r0 = s * wb_rows
            pltpu.sync_copy(acc.at[pl.ds(r0, wb_rows)],
                            outs_hbm.at[pl.ds(r0, wb_rows),
                                        pl.ds(c * H, H)])
            if with_counts:
                @pl.when(c == 0)
                def _():
                    pltpu.sync_copy(cacc.at[pl.ds(r0, wb_rows)],
                                    outc_hbm.at[pl.ds(r0, wb_rows)])

    return agg


@functools.lru_cache(maxsize=None)
def _make_dense(N, D, bf_out, R=1000):
    assert N % R == 0

    def body(x_ref, s_ref, c_ref, w_ref, b_ref, *o_refs):
        cnt = c_ref[:, 0:1]
        inv = jnp.where(cnt > 0.0, 1.0 / jnp.maximum(cnt, 1.0), 0.0)
        h = x_ref[...] + s_ref[...].astype(jnp.float32) * inv
        y = lax.dot_general(h, w_ref[...], (((1,), (1,)), ((), ())),
                            preferred_element_type=jnp.float32)
        y = jnp.maximum(y + b_ref[...], 0.0)
        o_refs[0][...] = y
        if bf_out:
            o_refs[1][...] = y.astype(jnp.bfloat16)

    n_out = 2 if bf_out else 1
    out_specs = [pl.BlockSpec((R, D), lambda i: (i, 0))] * n_out
    out_shape = [jax.ShapeDtypeStruct((N, D), jnp.float32)]
    if bf_out:
        out_shape.append(jax.ShapeDtypeStruct((N, D), jnp.bfloat16))

    return pl.pallas_call(
        body,
        grid=(N // R,),
        in_specs=[
            pl.BlockSpec((R, D), lambda i: (i, 0)),
            pl.BlockSpec((R, D), lambda i: (i, 0)),
            pl.BlockSpec((R, _CW), lambda i: (i, 0)),
            pl.BlockSpec((D, D), lambda i: (0, 0)),
            pl.BlockSpec((1, D), lambda i: (0, 0)),
        ],
        out_specs=out_specs,
        out_shape=out_shape,
    )


def kernel(x, edge_index, W1, b1, W2, b2):
    N, D = x.shape
    E = edge_index.shape[1]
    ept = E // _NS
    nch = ept // _C
    assert nch * _C == ept and nch % _K == 0

    src = edge_index[0].astype(jnp.int32).reshape(_NS, nch, _C)
    dst = edge_index[1].astype(jnp.int32).reshape(_NS, nch, _C)
    srcs = jnp.stack([2 * src, 2 * src + 1])

    zd = jnp.zeros((N, D // 2), jnp.bfloat16)
    zc = jnp.zeros((N, _CW), jnp.float32)
    ones = jnp.ones((_C, _CW), jnp.float32)

    agg_c = _make_agg(N, D, nch, True)
    agg = _make_agg(N, D, nch, False)
    dense_bf = _make_dense(N, D, True)
    dense = _make_dense(N, D, False)
    b1r = b1.reshape(1, D)
    b2r = b2.reshape(1, D)

    xb = x.astype(jnp.bfloat16)
    s1, c1 = agg_c(xb.reshape(2 * N, D // 2), srcs, dst, zd, zc, ones)
    h, hb = dense_bf(x, s1, c1, W1, b1r)
    (s2,) = agg(hb.reshape(2 * N, D // 2), srcs, dst, zd)
    (z,) = dense(h, s2, c1, W2, b2r)
    return z


# speedup vs baseline: 1.0100x; 1.0100x over previous
import functools

import jax
import jax.numpy as jnp
from jax import lax
from jax.experimental import pallas as pl
from jax.experimental.pallas import tpu as pltpu
from jax.experimental.pallas import tpu_sc as plsc

_NC = 2
_NS = 16
_C = 125
_CW = 8
_K = 8
_D4 = 4


@functools.lru_cache(maxsize=None)
def _make_agg(N, D, nch, with_counts):
    H = D // 2
    assert nch % _K == 0
    assert N % 80 == 0
    wb_rows = N // 10

    mesh = plsc.VectorSubcoreMesh(
        core_axis_name="c", subcore_axis_name="s",
        num_cores=_NC, num_subcores=_NS)

    out_type = [jax.ShapeDtypeStruct((N, D), jnp.bfloat16)]
    scratch = [
        pltpu.VMEM((nch, _C), jnp.int32),
        pltpu.VMEM((nch, _C), jnp.int32),
        pltpu.VMEM((_K, _C, H), jnp.bfloat16),
        pltpu.VMEM_SHARED((N, H), jnp.bfloat16),
    ] + [pltpu.SemaphoreType.DMA] * (2 * _K)
    if with_counts:
        out_type.append(jax.ShapeDtypeStruct((N, _CW), jnp.float32))
        scratch += [
            pltpu.VMEM((_C, _CW), jnp.float32),
            pltpu.VMEM_SHARED((N, _CW), jnp.float32),
            pltpu.SemaphoreType.DMA,
        ]

    @functools.partial(
        pl.kernel,
        out_type=tuple(out_type),
        mesh=mesh,
        scratch_types=scratch,
        compiler_params=pltpu.CompilerParams(use_tc_tiling_on_sc=False),
    )
    def agg(x2_hbm, src_hbm, dst_hbm, zd_hbm, *rest):
        if with_counts:
            (zc_hbm, ones_hbm, outs_hbm, outc_hbm,
             sidx, didx, ring, acc, *sems) = rest
            ones, cacc, csem = sems[-3:]
            sems = sems[:-3]
        else:
            (outs_hbm, sidx, didx, ring, acc, *sems) = rest
        gsem = sems[:_K]
        ssem = sems[_K:2 * _K]
        c = lax.axis_index("c")
        s = lax.axis_index("s")

        @pl.when(s == 0)
        def _():
            pltpu.sync_copy(zd_hbm, acc)
            if with_counts:
                @pl.when(c == 0)
                def _():
                    pltpu.sync_copy(zc_hbm, cacc)

        plsc.subcore_barrier()

        if with_counts:
            pltpu.sync_copy(ones_hbm, ones)

        pltpu.sync_copy(src_hbm.at[s], sidx)
        tbl = x2_hbm.at[pl.ds(c, 2 * N - 1)]
        pltpu.sync_copy(dst_hbm.at[s], didx)

        def fire_gather(m, b):
            pltpu.async_copy(tbl.at[sidx.at[m]], ring.at[b], gsem[b])

        def wait_gather(m, b):
            pltpu.make_async_copy(tbl.at[sidx.at[m]], ring.at[b],
                                  gsem[b]).wait()

        def fire_scatter(m, b):
            pltpu.async_copy(ring.at[b], acc.at[didx.at[m]], ssem[b],
                             add=True)

        def wait_scatter(m, b):
            pltpu.make_async_copy(ring.at[b], acc.at[didx.at[m]],
                                  ssem[b]).wait()

        for m in range(_D4):
            fire_gather(m, m)

        @pl.loop(0, nch, step=_K)
        def _(j):
            for o in range(_K):
                jj = j + o
                wait_gather(jj, o)
                fire_scatter(jj, o)
                if with_counts:
                    @pl.when(c == 0)
                    def _():
                        pltpu.async_copy(ones, cacc.at[didx.at[jj]], csem,
                                         add=True).wait()
                ob = (o + _D4) % _K
                @pl.when(jj >= _K - _D4)
                def _():
                    wait_scatter(jj - (_K - _D4), ob)

                @pl.when(jj + _D4 < nch)
                def _():
                    fire_gather(jj + _D4, ob)

        for m in range(nch - (_K - _D4), nch):
            wait_scatter(m, m % _K)

        plsc.subcore_barrier()

        @pl.when(s < 10)
        def _():
            r0 = s * wb_rows
            pltpu.sync_copy(acc.at[pl.ds(r0, wb_rows)],
                            outs_hbm.at[pl.ds(r0, wb_rows),
                                        pl.ds(c * H, H)])
            if with_counts:
                @pl.when(c == 0)
                def _():
                    pltpu.sync_copy(cacc.at[pl.ds(r0, wb_rows)],
                                    outc_hbm.at[pl.ds(r0, wb_rows)])

    return agg


@functools.lru_cache(maxsize=None)
def _make_dense(N, D, R=1000):
    assert N % R == 0

    def body(x_ref, s_ref, c_ref, w_ref, b_ref, o_ref):
        cnt = c_ref[:, 0:1]
        inv = jnp.where(cnt > 0.0, 1.0 / jnp.maximum(cnt, 1.0), 0.0)
        h = x_ref[...] + s_ref[...].astype(jnp.float32) * inv
        y = lax.dot_general(h, w_ref[...], (((1,), (1,)), ((), ())),
                            preferred_element_type=jnp.float32)
        o_ref[...] = jnp.maximum(y + b_ref[...], 0.0)

    out_specs = pl.BlockSpec((R, D), lambda i: (i, 0))
    out_shape = jax.ShapeDtypeStruct((N, D), jnp.float32)

    return pl.pallas_call(
        body,
        grid=(N // R,),
        in_specs=[
            pl.BlockSpec((R, D), lambda i: (i, 0)),
            pl.BlockSpec((R, D), lambda i: (i, 0)),
            pl.BlockSpec((R, _CW), lambda i: (i, 0)),
            pl.BlockSpec((D, D), lambda i: (0, 0)),
            pl.BlockSpec((1, D), lambda i: (0, 0)),
        ],
        out_specs=out_specs,
        out_shape=out_shape,
    )


def kernel(x, edge_index, W1, b1, W2, b2):
    N, D = x.shape
    E = edge_index.shape[1]
    ept = E // _NS
    nch = ept // _C
    assert nch * _C == ept and nch % _K == 0

    src = edge_index[0].astype(jnp.int32).reshape(_NS, nch, _C)
    dst = edge_index[1].astype(jnp.int32).reshape(_NS, nch, _C)
    srcs = 2 * src

    zd = jnp.zeros((N, D // 2), jnp.bfloat16)
    zc = jnp.zeros((N, _CW), jnp.float32)
    ones = jnp.ones((_C, _CW), jnp.float32)

    agg_c = _make_agg(N, D, nch, True)
    agg = _make_agg(N, D, nch, False)
    dense = _make_dense(N, D)
    b1r = b1.reshape(1, D)
    b2r = b2.reshape(1, D)

    xb = x.astype(jnp.bfloat16)
    s1, c1 = agg_c(xb.reshape(2 * N, D // 2), srcs, dst, zd, zc, ones)
    h = dense(x, s1, c1, W1, b1r)
    hb = h.astype(jnp.bfloat16)
    (s2,) = agg(hb.reshape(2 * N, D // 2), srcs, dst, zd)
    z = dense(h, s2, c1, W2, b2r)
    return z


# speedup vs baseline: 1.0424x; 1.0320x over previous
import functools

import jax
import jax.numpy as jnp
from jax import lax
from jax.experimental import pallas as pl
from jax.experimental.pallas import tpu as pltpu
from jax.experimental.pallas import tpu_sc as plsc

_NC = 2
_NS = 16
_C = 128
_CW = 8
_K = 6
_DA = 3


@functools.lru_cache(maxsize=None)
def _make_agg(N, D, blocks, with_counts):
    H = D // 2
    nbm = blocks // _NS
    nlo = blocks - nbm * _NS
    assert nbm % _K == 0 and nbm > _K and 0 <= nlo <= _NS
    assert N % 80 == 0
    wb_rows = N // 10

    mesh = plsc.VectorSubcoreMesh(
        core_axis_name="c", subcore_axis_name="s",
        num_cores=_NC, num_subcores=_NS)

    out_type = [jax.ShapeDtypeStruct((N, D), jnp.bfloat16)]
    scratch = [
        pltpu.VMEM((nbm + 1, 2, _C), jnp.int32),
        pltpu.VMEM((_K, _C, H), jnp.bfloat16),
        pltpu.VMEM_SHARED((N, H), jnp.bfloat16),
    ] + [pltpu.SemaphoreType.DMA] * (2 * _K)
    if with_counts:
        out_type.append(jax.ShapeDtypeStruct((N, _CW), jnp.float32))
        scratch += [
            pltpu.VMEM((_C, _CW), jnp.float32),
            pltpu.VMEM_SHARED((N, _CW), jnp.float32),
            pltpu.SemaphoreType.DMA,
        ]

    @functools.partial(
        pl.kernel,
        out_type=tuple(out_type),
        mesh=mesh,
        scratch_types=scratch,
        compiler_params=pltpu.CompilerParams(use_tc_tiling_on_sc=False),
    )
    def agg(x2_hbm, e_hbm, zd_hbm, *rest):
        if with_counts:
            (zc_hbm, ones_hbm, outs_hbm, outc_hbm,
             ei, ring, acc, *sems) = rest
            ones, cacc, csem = sems[-3:]
            sems = sems[:-3]
        else:
            (outs_hbm, ei, ring, acc, *sems) = rest
        gsem = sems[:_K]
        ssem = sems[_K:2 * _K]
        c = lax.axis_index("c")
        s = lax.axis_index("s")
        has_tail = s < nlo

        @pl.when(s == 0)
        def _():
            pltpu.sync_copy(zd_hbm, acc)
            if with_counts:
                @pl.when(c == 0)
                def _():
                    pltpu.sync_copy(zc_hbm, cacc)

        plsc.subcore_barrier()

        if with_counts:
            pltpu.sync_copy(ones_hbm, ones)

        pltpu.sync_copy(e_hbm.at[pl.ds(s * nbm, nbm)],
                        ei.at[pl.ds(0, nbm)])

        @pl.when(has_tail)
        def _():
            pltpu.sync_copy(e_hbm.at[pl.ds(_NS * nbm + s, 1)],
                            ei.at[pl.ds(nbm, 1)])

        @pl.loop(0, nbm + 1)
        def _(j):
            for k in range(_C // 16):
                sl = pl.ds(k * 16, 16)
                v = ei[j, 0, sl]
                ei[j, 0, sl] = v + v

        tbl = x2_hbm.at[pl.ds(c, 2 * N - 1)]

        def fire_gather(m, b):
            pltpu.async_copy(tbl.at[ei.at[m, 0]], ring.at[b], gsem[b])

        def wait_gather(m, b):
            pltpu.make_async_copy(tbl.at[ei.at[m, 0]], ring.at[b],
                                  gsem[b]).wait()

        def fire_scatter(m, b):
            pltpu.async_copy(ring.at[b], acc.at[ei.at[m, 1]], ssem[b],
                             add=True)

        def wait_scatter(m, b):
            pltpu.make_async_copy(ring.at[b], acc.at[ei.at[m, 1]],
                                  ssem[b]).wait()

        def count_edges(m):
            if with_counts:
                @pl.when(c == 0)
                def _():
                    pltpu.async_copy(ones, cacc.at[ei.at[m, 1]], csem,
                                     add=True).wait()

        for m in range(_DA):
            fire_gather(m, m)

        @pl.loop(0, nbm, step=_K)
        def _(j):
            for o in range(_K):
                jj = j + o
                wait_gather(jj, o)
                fire_scatter(jj, o)
                count_edges(jj)
                ob = (o + _DA) % _K
                @pl.when(jj >= _K - _DA)
                def _():
                    wait_scatter(jj - (_K - _DA), ob)

                @pl.when(jj + _DA < nbm)
                def _():
                    fire_gather(jj + _DA, ob)

        for m in range(nbm - (_K - _DA), nbm):
            wait_scatter(m, m % _K)

        @pl.when(has_tail)
        def _():
            fire_gather(nbm, 0)
            wait_gather(nbm, 0)
            fire_scatter(nbm, 0)
            count_edges(nbm)
            wait_scatter(nbm, 0)

        plsc.subcore_barrier()

        @pl.when(s < 10)
        def _():
            r0 = s * wb_rows
            pltpu.sync_copy(acc.at[pl.ds(r0, wb_rows)],
                            outs_hbm.at[pl.ds(r0, wb_rows),
                                        pl.ds(c * H, H)])
            if with_counts:
                @pl.when(c == 0)
                def _():
                    pltpu.sync_copy(cacc.at[pl.ds(r0, wb_rows)],
                                    outc_hbm.at[pl.ds(r0, wb_rows)])

    return agg


@functools.lru_cache(maxsize=None)
def _make_dense(N, D, R=1000):
    assert N % R == 0

    def body(x_ref, s_ref, c_ref, w_ref, b_ref, o_ref):
        cnt = c_ref[:, 0:1]
        inv = jnp.where(cnt > 0.0, 1.0 / jnp.maximum(cnt, 1.0), 0.0)
        h = x_ref[...] + s_ref[...].astype(jnp.float32) * inv
        y = lax.dot_general(h, w_ref[...], (((1,), (1,)), ((), ())),
                            preferred_element_type=jnp.float32)
        o_ref[...] = jnp.maximum(y + b_ref[...], 0.0)

    return pl.pallas_call(
        body,
        grid=(N // R,),
        in_specs=[
            pl.BlockSpec((R, D), lambda i: (i, 0)),
            pl.BlockSpec((R, D), lambda i: (i, 0)),
            pl.BlockSpec((R, _CW), lambda i: (i, 0)),
            pl.BlockSpec((D, D), lambda i: (0, 0)),
            pl.BlockSpec((1, D), lambda i: (0, 0)),
        ],
        out_specs=pl.BlockSpec((R, D), lambda i: (i, 0)),
        out_shape=jax.ShapeDtypeStruct((N, D), jnp.float32),
    )


def kernel(x, edge_index, W1, b1, W2, b2):
    N, D = x.shape
    E = edge_index.shape[1]
    blocks = E // _C
    assert blocks * _C == E

    ei = edge_index.astype(jnp.int32).reshape(2, blocks, _C).transpose(1, 0, 2)

    zd = jnp.zeros((N, D // 2), jnp.bfloat16)
    zc = jnp.zeros((N, _CW), jnp.float32)
    ones = jnp.ones((_C, _CW), jnp.float32)

    agg_c = _make_agg(N, D, blocks, True)
    agg = _make_agg(N, D, blocks, False)
    dense = _make_dense(N, D)
    b1r = b1.reshape(1, D)
    b2r = b2.reshape(1, D)

    xb = x.astype(jnp.bfloat16)
    s1, c1 = agg_c(xb.reshape(2 * N, D // 2), ei, zd, zc, ones)
    h = dense(x, s1, c1, W1, b1r)
    hb = h.astype(jnp.bfloat16)
    (s2,) = agg(hb.reshape(2 * N, D // 2), ei, zd)
    z = dense(h, s2, c1, W2, b2r)
    return z
